# baseline (speedup 1.0000x reference)
import jax
import jax.numpy as jnp
from jax import lax
from jax.experimental import pallas as pl
from jax.experimental.pallas import tpu as pltpu
from jax.experimental.pallas import tpu_sc as plsc

NC, NS, LANES = 2, 16, 16
NW = NC * NS
B, L, EMB = 4096, 200, 64
RPW = B // NW
SPLIT = 128
REST = L - SPLIT
NB = 4
NITER = RPW // NB
RU = 8
SCALE = 8.0

_mesh = plsc.VectorSubcoreMesh(core_axis_name="c", subcore_axis_name="s")


def _emb_body(tok_hbm, table_hbm, out_hbm, idx_v, *scratch):
    rows = scratch[:NB]
    gsems = scratch[NB:2 * NB]
    wsems = scratch[2 * NB:3 * NB]
    wid = lax.axis_index("s") * NC + lax.axis_index("c")
    rbase = wid * RPW
    pltpu.sync_copy(tok_hbm.at[pl.ds(rbase, RPW)], idx_v)

    def fire(r, b):
        pltpu.async_copy(
            table_hbm.at[idx_v.at[r, pl.ds(0, SPLIT)]],
            rows[b].at[pl.ds(0, SPLIT)], gsems[b])
        pltpu.async_copy(
            table_hbm.at[idx_v.at[r, pl.ds(SPLIT, REST)]],
            rows[b].at[pl.ds(SPLIT, REST)], gsems[b])

    def wait_gather(b):
        pltpu.make_async_copy(
            table_hbm.at[idx_v.at[0, pl.ds(0, SPLIT)]],
            rows[b].at[pl.ds(0, SPLIT)], gsems[b]).wait()
        pltpu.make_async_copy(
            table_hbm.at[idx_v.at[0, pl.ds(SPLIT, REST)]],
            rows[b].at[pl.ds(SPLIT, REST)], gsems[b]).wait()

    def wait_write(b):
        pltpu.make_async_copy(rows[b], out_hbm.at[rbase], wsems[b]).wait()

    for b in range(NB):
        fire(b, b)

    def scale_buf(buf):
        @plsc.parallel_loop(0, L, 1, unroll=RU)
        def _scale(r):
            for j in range(EMB // LANES):
                sl = pl.ds(j * LANES, LANES)
                buf[r, sl] = buf[r, sl] * SCALE

    def outer(it, carry):
        r0 = it * NB
        for b in range(NB):
            r = r0 + b
            wait_gather(b)
            scale_buf(rows[b])
            pltpu.async_copy(rows[b], out_hbm.at[rbase + r], wsems[b])

            @pl.when(r + NB < RPW)
            def _refire():
                wait_write(b)
                fire(r + NB, b)

        return carry

    lax.fori_loop(0, NITER, outer, 0)

    for b in range(NB):
        wait_write(b)


def kernel(tokens, embedding):
    return pl.kernel(
        _emb_body,
        mesh=_mesh,
        compiler_params=pltpu.CompilerParams(use_tc_tiling_on_sc=False),
        out_type=jax.ShapeDtypeStruct((B, L, EMB), jnp.float32),
        scratch_types=(
            [pltpu.VMEM((RPW, L), jnp.int32)]
            + [pltpu.VMEM((L, EMB), jnp.float32) for _ in range(NB)]
            + [pltpu.SemaphoreType.DMA for _ in range(2 * NB)]
        ),
    )(tokens, embedding)
